# Initial kernel scaffold; baseline (speedup 1.0000x reference)
import jax
import jax.numpy as jnp
from jax.experimental import pallas as pl


def kernel(idx, E1, E2, W1, b1, W2, b2):
    raise NotImplementedError("write your pallas kernel here")


# speedup vs baseline: 16.6439x; 16.6439x over previous
import jax
import jax.numpy as jnp
from jax.experimental import pallas as pl


K = 20
ROWS_PER_BLOCK = 256


def _nv_kernel(e1_ref, w1_ref, b1_ref, e2_ref, w2_ref, b2_ref,
               nv1_ref, nv2_ref):
    dn = (((1,), (1,)), ((), ()))
    x1 = jax.lax.dot_general(e1_ref[...], w1_ref[...], dn,
                             preferred_element_type=jnp.float32)
    x2 = jax.lax.dot_general(e2_ref[...], w2_ref[...], dn,
                             preferred_element_type=jnp.float32)
    nv1_ref[...] = jnp.tanh(x1 + b1_ref[...])
    nv2_ref[...] = jnp.tanh(x2 + b2_ref[...])


def _adj_kernel(x1_ref, x2_ref, nv1_ref, nv2_ref, out_ref):
    dn = (((1,), (1,)), ((), ()))
    a = jax.lax.dot_general(x1_ref[...], nv2_ref[...], dn,
                            preferred_element_type=jnp.float32)
    a -= jax.lax.dot_general(x2_ref[...], nv1_ref[...], dn,
                             preferred_element_type=jnp.float32)
    adj = jnp.maximum(jnp.tanh(a), 0.0)

    work = adj
    th = None
    for _ in range(K):
        th = jnp.max(work, axis=1, keepdims=True)
        work = jnp.where(work >= th, -1.0, work)
    out_ref[...] = jnp.where(adj >= th, adj, 0.0)


def kernel(idx, E1, E2, W1, b1, W2, b2):
    n = idx.shape[0]
    dim = E1.shape[1]
    e1 = jnp.take(E1, idx, axis=0)
    e2 = jnp.take(E2, idx, axis=0)

    nv1, nv2 = pl.pallas_call(
        _nv_kernel,
        out_shape=[
            jax.ShapeDtypeStruct((n, dim), jnp.float32),
            jax.ShapeDtypeStruct((n, dim), jnp.float32),
        ],
    )(e1, W1, b1.reshape(1, dim), e2, W2, b2.reshape(1, dim))

    r = min(ROWS_PER_BLOCK, n)
    out = pl.pallas_call(
        _adj_kernel,
        grid=(n // r,),
        in_specs=[
            pl.BlockSpec((r, dim), lambda i: (i, 0)),
            pl.BlockSpec((r, dim), lambda i: (i, 0)),
            pl.BlockSpec((n, dim), lambda i: (0, 0)),
            pl.BlockSpec((n, dim), lambda i: (0, 0)),
        ],
        out_specs=pl.BlockSpec((r, n), lambda i: (i, 0)),
        out_shape=jax.ShapeDtypeStruct((n, n), jnp.float32),
    )(nv1, nv2, nv1, nv2)
    return out


# speedup vs baseline: 17.1555x; 1.0307x over previous
import jax
import jax.numpy as jnp
from jax.experimental import pallas as pl


K = 20
ROWS_PER_BLOCK = 256


def _nv_kernel(e1_ref, w1_ref, b1_ref, e2_ref, w2_ref, b2_ref,
               nv1_ref, nv2_ref):
    dn = (((1,), (1,)), ((), ()))
    x1 = jax.lax.dot_general(e1_ref[...], w1_ref[...], dn,
                             preferred_element_type=jnp.float32)
    x2 = jax.lax.dot_general(e2_ref[...], w2_ref[...], dn,
                             preferred_element_type=jnp.float32)
    nv1_ref[...] = jnp.tanh(x1 + b1_ref[...])
    nv2_ref[...] = jnp.tanh(x2 + b2_ref[...])


def _adj_kernel(x1_ref, x2_ref, nv1_ref, nv2_ref, out_ref):
    dn = (((1,), (1,)), ((), ()))
    lhs = jnp.concatenate([x1_ref[...], -x2_ref[...]], axis=1)
    rhs = jnp.concatenate([nv2_ref[...], nv1_ref[...]], axis=1)
    a = jax.lax.dot_general(lhs, rhs, dn, preferred_element_type=jnp.float32)
    adj = jnp.maximum(jnp.tanh(a), 0.0)

    work = adj
    th = None
    for _ in range(K):
        th = jnp.max(work, axis=1, keepdims=True)
        work = jnp.where(work >= th, -1.0, work)
    out_ref[...] = jnp.where(adj >= th, adj, 0.0)


def kernel(idx, E1, E2, W1, b1, W2, b2):
    n = idx.shape[0]
    dim = E1.shape[1]
    e1 = jnp.take(E1, idx, axis=0)
    e2 = jnp.take(E2, idx, axis=0)

    nv1, nv2 = pl.pallas_call(
        _nv_kernel,
        out_shape=[
            jax.ShapeDtypeStruct((n, dim), jnp.float32),
            jax.ShapeDtypeStruct((n, dim), jnp.float32),
        ],
    )(e1, W1, b1.reshape(1, dim), e2, W2, b2.reshape(1, dim))

    r = min(ROWS_PER_BLOCK, n)
    out = pl.pallas_call(
        _adj_kernel,
        grid=(n // r,),
        in_specs=[
            pl.BlockSpec((r, dim), lambda i: (i, 0)),
            pl.BlockSpec((r, dim), lambda i: (i, 0)),
            pl.BlockSpec((n, dim), lambda i: (0, 0)),
            pl.BlockSpec((n, dim), lambda i: (0, 0)),
        ],
        out_specs=pl.BlockSpec((r, n), lambda i: (i, 0)),
        out_shape=jax.ShapeDtypeStruct((n, n), jnp.float32),
    )(nv1, nv2, nv1, nv2)
    return out
